# Initial kernel scaffold; baseline (speedup 1.0000x reference)
import jax
import jax.numpy as jnp
from jax.experimental import pallas as pl


def kernel(x, edge_index, We1, be1, We2, be2, Wg1, bg1, Wg2, bg2, Wa1, as1, ad1, ba1, Wa2, as2, ad2, ba2, Wp1, bp1, Wp2, bp2, Wd1, bd1, g1, bt1, Wd2, bd2, g2, bt2, Wd3, bd3):
    raise NotImplementedError("write your pallas kernel here")


# speedup vs baseline: 4.1502x; 4.1502x over previous
import functools

import jax
import jax.numpy as jnp
from jax import lax
from jax.experimental import pallas as pl
from jax.experimental.pallas import tpu as pltpu
from jax.experimental.pallas import tpu_sc as plsc

f32 = jnp.float32
i32 = jnp.int32

N_ = 10000
D_ = 128
H_ = 8
CH_ = 16
NC, NS, L = 2, 16, 16
NW = NC * NS
C = 128
NP_ = 10112
CE = 512


def _mesh():
    return plsc.VectorSubcoreMesh(
        core_axis_name="c", subcore_axis_name="s", num_cores=NC, num_subcores=NS
    )


def _lrelu(v):
    return jnp.maximum(v, 0.0) + 0.2 * jnp.minimum(v, 0.0)


def _ew_gather_kernel(Ep, K, Epw):
    @functools.partial(
        pl.kernel,
        out_type=[jax.ShapeDtypeStruct((Ep, D_), f32)],
        mesh=_mesh(),
        scratch_types=[
            pltpu.VMEM((C,), i32),
            pltpu.VMEM((C,), i32),
            pltpu.VMEM((C, D_), f32),
            pltpu.VMEM((C, D_), f32),
            pltpu.SemaphoreType.DMA,
            pltpu.SemaphoreType.DMA,
        ],
    )
    def k(a_h, b_h, rowg_h, colg_h, g_h, idxr, idxc, abuf, bbuf, s1, s2):
        c = lax.axis_index("c")
        s = lax.axis_index("s")
        wid = s * NC + c
        e0 = wid * Epw

        def chunk(icb, _):
            eb = e0 + icb * C
            pltpu.sync_copy(rowg_h.at[pl.ds(eb, C)], idxr)
            pltpu.sync_copy(colg_h.at[pl.ds(eb, C)], idxc)
            cp1 = pltpu.async_copy(a_h.at[idxr], abuf, s1)
            cp2 = pltpu.async_copy(b_h.at[idxc], bbuf, s2)
            cp1.wait()
            cp2.wait()

            def edge(e, _):
                for kk in range(D_ // L):
                    abuf[e, pl.ds(kk * L, L)] = (
                        abuf[e, pl.ds(kk * L, L)] + bbuf[e, pl.ds(kk * L, L)])
                return 0

            lax.fori_loop(0, C, edge, 0)
            pltpu.sync_copy(abuf, g_h.at[pl.ds(eb, C)])
            return 0

        lax.fori_loop(0, K, chunk, 0)

    return k


def _pair16_kernel(Ep, K, Epw):
    @functools.partial(
        pl.kernel,
        out_type=[jax.ShapeDtypeStruct((Ep, L), f32)],
        mesh=_mesh(),
        scratch_types=[
            pltpu.VMEM((C,), i32),
            pltpu.VMEM((C,), i32),
            pltpu.VMEM((C, D_), f32),
            pltpu.VMEM((C, D_), f32),
            pltpu.VMEM((C, L), f32),
            pltpu.SemaphoreType.DMA,
            pltpu.SemaphoreType.DMA,
        ],
    )
    def k(t_h, rowg_h, colg_h, av_h, idxr, idxc, rbuf, cbuf, amat, s1, s2):
        c = lax.axis_index("c")
        s = lax.axis_index("s")
        wid = s * NC + c
        e0 = wid * Epw

        def chunk(icb, _):
            eb = e0 + icb * C
            pltpu.sync_copy(rowg_h.at[pl.ds(eb, C)], idxr)
            pltpu.sync_copy(colg_h.at[pl.ds(eb, C)], idxc)
            cp1 = pltpu.async_copy(t_h.at[idxr], rbuf, s1)
            cp2 = pltpu.async_copy(t_h.at[idxc], cbuf, s2)
            cp1.wait()
            cp2.wait()

            def edge(e, _):
                amat[e, :] = rbuf[e, pl.ds(0, L)] + cbuf[e, pl.ds(L, L)]
                return 0

            lax.fori_loop(0, C, edge, 0)
            pltpu.sync_copy(amat, av_h.at[pl.ds(eb, C)])
            return 0

        lax.fori_loop(0, K, chunk, 0)

    return k


def _scale_kernel(Ep, K, Epw):
    @functools.partial(
        pl.kernel,
        out_type=[jax.ShapeDtypeStruct((Ep, D_), f32)],
        mesh=_mesh(),
        scratch_types=[
            pltpu.VMEM((C,), i32),
            pltpu.VMEM((C, D_), f32),
            pltpu.VMEM((C,), f32),
            pltpu.SemaphoreType.DMA,
        ],
    )
    def k(h_h, ew_h, rowg_h, val_h, idxr, hbuf, ebuf, s1):
        c = lax.axis_index("c")
        s = lax.axis_index("s")
        wid = s * NC + c
        e0 = wid * Epw

        def chunk(icb, _):
            eb = e0 + icb * C
            pltpu.sync_copy(rowg_h.at[pl.ds(eb, C)], idxr)
            pltpu.sync_copy(ew_h.at[pl.ds(eb, C)], ebuf)
            cp = pltpu.async_copy(h_h.at[idxr], hbuf, s1)
            cp.wait()

            def grp(g, _):
                ewv = ebuf[pl.ds(g * L, L)]
                for j in range(L):
                    e = g * L + j
                    nj = jnp.broadcast_to(ewv[j], (L,))
                    for kk in range(D_ // L):
                        hbuf[e, pl.ds(kk * L, L)] = (
                            hbuf[e, pl.ds(kk * L, L)] * nj)
                return 0

            lax.fori_loop(0, C // L, grp, 0)
            pltpu.sync_copy(hbuf, val_h.at[pl.ds(eb, C)])
            return 0

        lax.fori_loop(0, K, chunk, 0)

    return k


def _gatt_kernel(Ep, K, Epw):
    @functools.partial(
        pl.kernel,
        out_type=[jax.ShapeDtypeStruct((Ep, D_), f32)],
        mesh=_mesh(),
        scratch_types=[
            pltpu.VMEM((C,), i32),
            pltpu.VMEM((C,), i32),
            pltpu.VMEM((C, D_), f32),
            pltpu.VMEM((C, D_), f32),
            pltpu.VMEM((C, L), f32),
            pltpu.VMEM((C, L), f32),
            pltpu.SemaphoreType.DMA,
            pltpu.SemaphoreType.DMA,
        ],
    )
    def k(ha_h, t2_h, ex_h, rowg_h, colg_h, val_h,
          idxr, idxc, hbuf, ibuf, exm, attm, s1, s2):
        c = lax.axis_index("c")
        s = lax.axis_index("s")
        wid = s * NC + c
        e0 = wid * Epw

        def chunk(icb, _):
            eb = e0 + icb * C
            pltpu.sync_copy(rowg_h.at[pl.ds(eb, C)], idxr)
            pltpu.sync_copy(colg_h.at[pl.ds(eb, C)], idxc)
            pltpu.sync_copy(ex_h.at[pl.ds(eb, C)], exm)
            cp1 = pltpu.async_copy(ha_h.at[idxr], hbuf, s1)
            cp2 = pltpu.async_copy(t2_h.at[idxc], ibuf, s2)
            cp1.wait()
            cp2.wait()

            def prep(e, _):
                attm[e, :] = exm[e, :] * ibuf[e, pl.ds(0, L)]
                return 0

            lax.fori_loop(0, C, prep, 0)

            def edge(e, _):
                atr = attm[e, :]
                for hd in range(H_):
                    ah = jnp.broadcast_to(atr[hd], (CH_,))
                    hbuf[e, pl.ds(hd * CH_, CH_)] = (
                        hbuf[e, pl.ds(hd * CH_, CH_)] * ah)
                return 0

            lax.fori_loop(0, C, edge, 0)
            pltpu.sync_copy(hbuf, val_h.at[pl.ds(eb, C)])
            return 0

        lax.fori_loop(0, K, chunk, 0)

    return k


def _segsum(Ep, W, vals, cols2):
    def body(cols_r, val_r, out_r):
        @pl.when(pl.program_id(0) == 0)
        def _():
            out_r[...] = jnp.zeros((NP_, W), f32)

        def edge(e, _):
            cc = cols_r[0, e]
            out_r[pl.ds(cc, 1), :] = (out_r[pl.ds(cc, 1), :]
                                      + val_r[pl.ds(e, 1), :])
            return 0

        lax.fori_loop(0, CE, edge, 0)

    return pl.pallas_call(
        body,
        grid=(Ep // CE,),
        in_specs=[
            pl.BlockSpec((1, CE), lambda i: (0, i),
                         memory_space=pltpu.SMEM),
            pl.BlockSpec((CE, W), lambda i: (i, 0)),
        ],
        out_specs=pl.BlockSpec((NP_, W), lambda i: (0, 0)),
        out_shape=jax.ShapeDtypeStruct((NP_, W), f32),
    )(cols2, vals)


def _mt(a, w):
    return lax.dot_general(a, w, (((1,), (1,)), ((), ())),
                           preferred_element_type=f32)


def _mm(a, w):
    return lax.dot_general(a, w, (((1,), (0,)), ((), ())),
                           preferred_element_type=f32)


def _tc(fn, out_shapes):
    return pl.pallas_call(fn, out_shape=out_shapes)


def _t0(x_r, we1_r, be1_r, wg1_r, a_r, b_r, h1_r):
    x = x_r[...]
    we1 = we1_r[...]
    c0 = we1[:, 2 * D_] + be1_r[...]
    a_r[...] = _mt(x, we1[:, :D_]) + c0
    b_r[...] = _mt(x, we1[:, D_:2 * D_])
    h1_r[...] = _mt(x, wg1_r[...])


def _tew(g_r, w2_r, be2_r, ew_r):
    z = _mm(jnp.maximum(g_r[...], 0.0), w2_r[...]) + be2_r[...]
    ew_r[...] = 1.0 / (1.0 + jnp.exp(-z))


def _t1(deg_r, h1_r, dis_r, sw_r, h1p_r):
    deg = 1.0 + deg_r[:, :1]
    dis = lax.rsqrt(deg)
    dis_r[...] = dis
    sw_r[...] = 1.0 / deg
    h1p_r[...] = dis[:N_, :] * h1_r[...]


def _t2(agg_r, h1_r, dis_r, sw_r, bg1_r, wg2_r, h2_r, h2p_r):
    dis = dis_r[:N_, :]
    x1 = jnp.maximum(dis * agg_r[:N_, :] + sw_r[:N_, :] * h1_r[...]
                     + bg1_r[...], 0.0)
    h2 = _mt(x1, wg2_r[...])
    h2_r[...] = h2
    h2p_r[...] = dis * h2


def _gat_tables(ha, asm, adm):
    asrc = _mm(ha, asm)
    adst = _mm(ha, adm)
    al = asrc + adst
    exs = jnp.exp(jnp.where(al >= 0, al, 0.2 * al))
    zpad8 = jnp.zeros((N_, H_), f32)
    zpad96 = jnp.zeros((N_, D_ - 4 * H_), f32)
    tt = jnp.concatenate([asrc, zpad8, adst, zpad8, zpad96], axis=1)
    return tt, exs


def _t3(agg_r, h2_r, dis_r, sw_r, bg2_r, wa_r, asm_r, adm_r,
        ha_r, t_r, exs_r):
    x1 = jnp.maximum(dis_r[:N_, :] * agg_r[:N_, :]
                     + sw_r[:N_, :] * h2_r[...] + bg2_r[...], 0.0)
    ha = _mt(x1, wa_r[...])
    tt, exs = _gat_tables(ha, asm_r[...], adm_r[...])
    ha_r[...] = ha
    t_r[...] = tt
    exs_r[...] = exs


def _tex(av_r, ex_r):
    ex_r[...] = jnp.exp(_lrelu(av_r[...]))


def _t4(den_r, exs_r, t2_r, sa_r):
    den = den_r[:N_, :H_] + exs_r[...]
    inv = 1.0 / (den + 1e-16)
    t2_r[...] = jnp.concatenate(
        [inv, jnp.zeros((N_, H_), f32), jnp.zeros((N_, D_ - 2 * H_), f32)],
        axis=1)
    sa_r[...] = exs_r[...] * inv


def _t5(agg_r, ha1_r, sa_r, p_r, ba_r, wa2_r, asm_r, adm_r,
        ha_r, t_r, exs_r):
    sb = _mm(sa_r[...], p_r[...])
    x2 = agg_r[:N_, :] + ha1_r[...] * sb + ba_r[...]
    ha = _mt(x2, wa2_r[...])
    tt, exs = _gat_tables(ha, asm_r[...], adm_r[...])
    ha_r[...] = ha
    t_r[...] = tt
    exs_r[...] = exs


def _t7(agg_r, ha2_r, sa_r, p_r, ba_r, wp1_r, dis_r, h3_r, h3p_r, x2m_r):
    sb = _mm(sa_r[...], p_r[...])
    x2 = agg_r[:N_, :] + ha2_r[...] * sb + ba_r[...]
    x2m_r[...] = jnp.mean(x2, axis=0, keepdims=True)
    h3 = _mt(x2, wp1_r[...])
    h3_r[...] = h3
    h3p_r[...] = jnp.concatenate(
        [dis_r[:N_, :] * h3, jnp.zeros((N_, D_ - 64), f32)], axis=1)


def _t8(agg_r, h3_r, dis_r, sw_r, bp1_r, wp2_r, h4_r, h4p_r, x3m_r):
    dis = dis_r[:N_, :]
    x3 = jnp.maximum(dis * agg_r[:N_, :64] + sw_r[:N_, :] * h3_r[...]
                     + bp1_r[...], 0.0)
    x3m_r[...] = jnp.mean(x3, axis=0, keepdims=True)
    h4 = _mt(x3, wp2_r[...])
    h4_r[...] = h4
    h4p_r[...] = jnp.concatenate(
        [dis * h4, jnp.zeros((N_, D_ - 32), f32)], axis=1)


def _ln(v, g, b):
    m = jnp.mean(v, axis=-1, keepdims=True)
    var = jnp.mean((v - m) ** 2, axis=-1, keepdims=True)
    return (v - m) / jnp.sqrt(var + 1e-5) * g + b


def _t9(agg_r, h4_r, dis_r, sw_r, bp2_r, x2m_r, x3m_r,
        wd1_r, bd1_r, g1_r, bt1_r, wd2_r, bd2_r, g2_r, bt2_r, wd3_r, bd3_r,
        out_r):
    x4 = jnp.maximum(dis_r[:N_, :] * agg_r[:N_, :32]
                     + sw_r[:N_, :] * h4_r[...] + bp2_r[...], 0.0)
    x4m = jnp.max(x4, axis=0, keepdims=True)
    wd1 = wd1_r[...]
    v = (_mt(x2m_r[...], wd1[:, :D_]) + _mt(x3m_r[...], wd1[:, D_:D_ + 64])
         + _mt(x4m, wd1[:, D_ + 64:]) + bd1_r[...])
    v = jnp.maximum(_ln(v, g1_r[...], bt1_r[...]), 0.0)
    v = jnp.maximum(_ln(_mt(v, wd2_r[...]) + bd2_r[...], g2_r[...],
                        bt2_r[...]), 0.0)
    out_r[...] = _mt(v, wd3_r[...]) + bd3_r[...]


def kernel(x, edge_index, We1, be1, We2, be2, Wg1, bg1, Wg2, bg2,
           Wa1, as1, ad1, ba1, Wa2, as2, ad2, ba2, Wp1, bp1, Wp2, bp2,
           Wd1, bd1, g1, bt1, Wd2, bd2, g2, bt2, Wd3, bd3):
    n = x.shape[0]
    e = edge_index.shape[1]
    K = -(-e // (NW * C))
    Epw = K * C
    Ep = NW * Epw
    pad = Ep - e

    row = edge_index[0]
    col = edge_index[1]
    zpad = jnp.zeros((pad,), i32)
    rowg = jnp.concatenate([row, zpad])
    colg = jnp.concatenate([col, zpad])
    cols2 = jnp.concatenate([col, jnp.full((pad,), n, i32)]).reshape(1, Ep)

    eye = jnp.eye(H_, dtype=f32)
    as1m = (eye[:, None, :] * as1[:, :, None]).reshape(D_, H_)
    ad1m = (eye[:, None, :] * ad1[:, :, None]).reshape(D_, H_)
    as2m = (eye[:, None, :] * as2[:, :, None]).reshape(D_, H_)
    ad2m = (eye[:, None, :] * ad2[:, :, None]).reshape(D_, H_)
    pm = jnp.repeat(eye, CH_, axis=1)
    w2c = We2.reshape(D_, 1)
    be2c = be2.reshape(1, 1)

    sds = jax.ShapeDtypeStruct

    A, B, h1 = _tc(_t0, [sds((n, D_), f32)] * 3)(x, We1, be1, Wg1)
    (g_e,) = _ew_gather_kernel(Ep, K, Epw)(A, B, rowg, colg)
    BR = 4096
    ew2 = pl.pallas_call(
        _tew, grid=(Ep // BR,),
        in_specs=[pl.BlockSpec((BR, D_), lambda i: (i, 0)),
                  pl.BlockSpec((D_, 1), lambda i: (0, 0)),
                  pl.BlockSpec((1, 1), lambda i: (0, 0))],
        out_specs=pl.BlockSpec((BR, 1), lambda i: (i, 0)),
        out_shape=sds((Ep, 1), f32),
    )(g_e, w2c, be2c)
    ew_p = ew2.reshape(Ep)

    deg = _segsum(Ep, 1, ew2, cols2)
    dis2, sw2, h1p = _tc(
        _t1, [sds((NP_, 1), f32), sds((NP_, 1), f32), sds((n, D_), f32)])(
        deg, h1)

    scale = _scale_kernel(Ep, K, Epw)
    gatt = _gatt_kernel(Ep, K, Epw)
    pair = _pair16_kernel(Ep, K, Epw)

    (v1,) = scale(h1p, ew_p, rowg)
    agg1 = _segsum(Ep, D_, v1, cols2)
    h2, h2p = _tc(_t2, [sds((n, D_), f32)] * 2)(agg1, h1, dis2, sw2, bg1, Wg2)
    (v2,) = scale(h2p, ew_p, rowg)
    agg2 = _segsum(Ep, D_, v2, cols2)

    def gat_layer(tc_fn, tc_args, asm, adm):
        ha, tt, exs = _tc(
            tc_fn, [sds((n, D_), f32), sds((n, D_), f32),
                    sds((n, H_), f32)])(*tc_args)
        (av,) = pair(tt, rowg, colg)
        ex = pl.pallas_call(
            _tex, grid=(Ep // BR,),
            in_specs=[pl.BlockSpec((BR, L), lambda i: (i, 0))],
            out_specs=pl.BlockSpec((BR, L), lambda i: (i, 0)),
            out_shape=sds((Ep, L), f32),
        )(av)
        den = _segsum(Ep, L, ex, cols2)
        t2t, sa = _tc(_t4, [sds((n, D_), f32), sds((n, H_), f32)])(den, exs)
        (vg,) = gatt(ha, t2t, ex, rowg, colg)
        aggb = _segsum(Ep, D_, vg, cols2)
        return ha, sa, aggb

    ha1, sa1, aggb1 = gat_layer(
        _t3, (agg2, h2, dis2, sw2, bg2, Wa1, as1m, ad1m), as1m, ad1m)
    ha2, sa2, aggb2 = gat_layer(
        _t5, (aggb1, ha1, sa1, pm, ba1, Wa2, as2m, ad2m), as2m, ad2m)

    h3, h3p, x2m = _tc(
        _t7, [sds((n, 64), f32), sds((n, D_), f32), sds((1, D_), f32)])(
        aggb2, ha2, sa2, pm, ba2, Wp1, dis2)
    (v3,) = scale(h3p, ew_p, rowg)
    agg3 = _segsum(Ep, D_, v3, cols2)
    h4, h4p, x3m = _tc(
        _t8, [sds((n, 32), f32), sds((n, D_), f32), sds((1, 64), f32)])(
        agg3, h3, dis2, sw2, bp1, Wp2)
    (v4,) = scale(h4p, ew_p, rowg)
    agg4 = _segsum(Ep, D_, v4, cols2)
    out = _tc(_t9, [sds((1, D_), f32)])(
        agg4, h4, dis2, sw2, bp2, x2m, x3m,
        Wd1, bd1, g1, bt1, Wd2, bd2, g2, bt2, Wd3, bd3)
    return out[0]
